# baseline (speedup 1.0000x reference)
import functools

import jax
import jax.numpy as jnp
from jax import lax
from jax.experimental import pallas as pl
from jax.experimental.pallas import tpu as pltpu
from jax.experimental.pallas import tpu_sc as plsc

_NUM_CORES = 2
_NUM_SUBCORES = 16
_NW = _NUM_CORES * _NUM_SUBCORES
_CHUNK = 512


@functools.partial(jax.jit, static_argnums=(2, 3))
def _embed_gather(x_flat, table, n, d):
    per_w = n // _NW
    nchunk = per_w // _CHUNK

    mesh = plsc.VectorSubcoreMesh(
        core_axis_name="c", subcore_axis_name="s")

    @functools.partial(
        pl.kernel,
        out_type=jax.ShapeDtypeStruct((n, d), jnp.float32),
        mesh=mesh,
        scratch_types=[
            pltpu.VMEM((_CHUNK,), jnp.int32),
            pltpu.VMEM((_CHUNK, d), jnp.float32),
            pltpu.SemaphoreType.DMA,
        ],
        compiler_params=pltpu.CompilerParams(use_tc_tiling_on_sc=False),
    )
    def gather_kernel(idx_hbm, table_hbm, out_hbm, idx_v, rows_v, sem):
        wid = lax.axis_index("s") * _NUM_CORES + lax.axis_index("c")
        base = wid * per_w

        def body(i, _):
            off = base + i * _CHUNK
            pltpu.sync_copy(idx_hbm.at[pl.ds(off, _CHUNK)], idx_v)
            pltpu.async_copy(table_hbm.at[idx_v], rows_v, sem).wait()
            pltpu.sync_copy(rows_v, out_hbm.at[pl.ds(off, _CHUNK)])
            return ()

        lax.fori_loop(0, nchunk, body, ())

    return gather_kernel(x_flat, table)


def kernel(x, table):
    b, l = x.shape
    _, d = table.shape
    n = b * l
    out = _embed_gather(x.reshape(n).astype(jnp.int32), table, n, d)
    return out.reshape(b, l, d)


# speedup vs baseline: 1.0448x; 1.0448x over previous
import functools

import jax
import jax.numpy as jnp
from jax import lax
from jax.experimental import pallas as pl
from jax.experimental.pallas import tpu as pltpu
from jax.experimental.pallas import tpu_sc as plsc

_NUM_CORES = 2
_NUM_SUBCORES = 16
_NW = _NUM_CORES * _NUM_SUBCORES
_CHUNK = 512
_NBUF = 2


@functools.partial(jax.jit, static_argnums=(2, 3))
def _embed_gather(x_flat, table, n, d):
    per_w = n // _NW
    nchunk = per_w // _CHUNK
    assert nchunk % _NBUF == 0

    mesh = plsc.VectorSubcoreMesh(
        core_axis_name="c", subcore_axis_name="s")

    @functools.partial(
        pl.kernel,
        out_type=jax.ShapeDtypeStruct((n, d), jnp.float32),
        mesh=mesh,
        scratch_types=[
            pltpu.VMEM((_CHUNK,), jnp.int32),
            pltpu.VMEM((_CHUNK,), jnp.int32),
            pltpu.VMEM((_CHUNK, d), jnp.float32),
            pltpu.VMEM((_CHUNK, d), jnp.float32),
            pltpu.SemaphoreType.DMA,
            pltpu.SemaphoreType.DMA,
            pltpu.SemaphoreType.DMA,
            pltpu.SemaphoreType.DMA,
            pltpu.SemaphoreType.DMA,
            pltpu.SemaphoreType.DMA,
        ],
        compiler_params=pltpu.CompilerParams(use_tc_tiling_on_sc=False),
    )
    def gather_kernel(idx_hbm, table_hbm, out_hbm, i0, i1, r0, r1,
                      is0, is1, g0, g1, s0, s1):
        wid = lax.axis_index("s") * _NUM_CORES + lax.axis_index("c")
        base = wid * per_w
        ibuf = [i0, i1]
        rows = [r0, r1]
        isem = [is0, is1]
        gsem = [g0, g1]
        ssem = [s0, s1]

        def idx_src(j):
            return idx_hbm.at[pl.ds(base + j * _CHUNK, _CHUNK)]

        def out_slice(j):
            return out_hbm.at[pl.ds(base + j * _CHUNK, _CHUNK)]

        for b in range(_NBUF):
            pltpu.async_copy(idx_src(b), ibuf[b], isem[b])
        for b in range(_NBUF):
            pltpu.make_async_copy(idx_src(b), ibuf[b], isem[b]).wait()
            pltpu.async_copy(table_hbm.at[ibuf[b]], rows[b], gsem[b])

        def body(i, _):
            for b in range(_NBUF):
                j = i + b
                pltpu.make_async_copy(
                    table_hbm.at[ibuf[b]], rows[b], gsem[b]).wait()

                @pl.when(j + _NBUF < nchunk)
                def _():
                    pltpu.async_copy(
                        idx_src(j + _NBUF), ibuf[b], isem[b])

                pltpu.async_copy(rows[b], out_slice(j), ssem[b])

                @pl.when(j + _NBUF < nchunk)
                def _():
                    pltpu.make_async_copy(
                        rows[b], out_slice(j), ssem[b]).wait()
                    pltpu.make_async_copy(
                        idx_src(j + _NBUF), ibuf[b], isem[b]).wait()
                    pltpu.async_copy(
                        table_hbm.at[ibuf[b]], rows[b], gsem[b])
            return ()

        lax.fori_loop(0, nchunk // _NBUF, lambda i, c: body(i * _NBUF, c), (),
                      unroll=False)

        for b in range(_NBUF):
            j = nchunk - _NBUF + b
            pltpu.make_async_copy(rows[b], out_slice(j), ssem[b]).wait()

    return gather_kernel(x_flat, table)


def kernel(x, table):
    b, l = x.shape
    _, d = table.shape
    n = b * l
    out = _embed_gather(x.reshape(n).astype(jnp.int32), table, n, d)
    return out.reshape(b, l, d)
